# Initial kernel scaffold; baseline (speedup 1.0000x reference)
import jax
import jax.numpy as jnp
from jax.experimental import pallas as pl


def kernel(x, edge_index, W_in, b_in, W_self0, W_neigh0, W_self1, W_neigh1, W_out, b_out):
    raise NotImplementedError("write your pallas kernel here")


# speedup vs baseline: 134.2882x; 134.2882x over previous
import functools

import jax
import jax.numpy as jnp
import numpy as np
from jax import lax
from jax.experimental import pallas as pl
from jax.experimental.pallas import tpu as pltpu
from jax.experimental.pallas import tpu_sc as plsc

N = 10000
E = 320000
D = 128
K = 32
OUT = 128

NCORES = 2
NSUB = 16
NW = NCORES * NSUB
WIN = 128
EP = ((E + NW * WIN - 1) // (NW * WIN)) * (NW * WIN)
EPW = EP // NW
NWIN = EPW // WIN
RPT = 632
NPAD = RPT * NSUB

ROWS_BLK = 1000
NBLK = N // ROWS_BLK


def _maxk_block(t):
    b = lax.bitcast_convert_type(t, jnp.int32)
    neg = lax.shift_right_arithmetic(b, jnp.int32(31))
    key = lax.bitwise_xor(b, lax.bitwise_and(neg, jnp.int32(0x7FFFFFFF)))
    sign = jnp.int32(-2147483648)
    t_u = jnp.zeros(t.shape[:1] + (1,), jnp.int32)
    for bit in range(31, -1, -1):
        cand = lax.bitwise_or(t_u, jnp.int32(np.int32(np.uint32(1 << bit))))
        thr = lax.bitwise_xor(cand, sign)
        cnt = jnp.sum((key >= thr).astype(jnp.int32), axis=1, keepdims=True,
                      dtype=jnp.int32)
        t_u = jnp.where(cnt >= K, cand, t_u)
    thresh = lax.bitwise_xor(t_u, sign)
    return t * (key >= thresh).astype(t.dtype)


def _dot_t(a, w):
    return lax.dot_general(a, w, (((1,), (1,)), ((), ())),
                           precision=lax.Precision.HIGHEST,
                           preferred_element_type=jnp.float32)


def _tc_in_body(x_ref, w_ref, b_ref, o_ref):
    o_ref[...] = _maxk_block(_dot_t(x_ref[...], w_ref[...]) + b_ref[...])


def _hagg(a0_ref, a1_ref, d0_ref, d1_ref):
    deg = d0_ref[...][:, 0:1] + d1_ref[...][:, 0:1]
    return (a0_ref[...] + a1_ref[...]) / (deg + jnp.float32(1e-6))


def _tc_mid_body(hs_ref, a0_ref, a1_ref, d0_ref, d1_ref, ws_ref, wn_ref, o_ref):
    t = _dot_t(hs_ref[...], ws_ref[...]) + _dot_t(
        _hagg(a0_ref, a1_ref, d0_ref, d1_ref), wn_ref[...])
    o_ref[...] = _maxk_block(t)


def _tc_out_body(hs_ref, a0_ref, a1_ref, d0_ref, d1_ref, ws_ref, wn_ref,
                 wo_ref, bo_ref, o_ref):
    t = _dot_t(hs_ref[...], ws_ref[...]) + _dot_t(
        _hagg(a0_ref, a1_ref, d0_ref, d1_ref), wn_ref[...])
    o_ref[...] = _dot_t(t, wo_ref[...]) + bo_ref[...]


def _row_spec():
    return pl.BlockSpec((ROWS_BLK, D), lambda i: (i, jnp.int32(0)))


def _deg_spec():
    return pl.BlockSpec((ROWS_BLK, 16), lambda i: (i, jnp.int32(0)))


def _full_spec(r, c):
    return pl.BlockSpec((r, c), lambda i: (jnp.int32(0), jnp.int32(0)))


def _tc_in(x, w, b):
    return pl.pallas_call(
        _tc_in_body,
        grid=(NBLK,),
        in_specs=[_row_spec(), _full_spec(D, D), _full_spec(1, D)],
        out_specs=_row_spec(),
        out_shape=jax.ShapeDtypeStruct((N, D), jnp.float32),
    )(x, w, b)


def _tc_mid(hs, a0, a1, d0, d1, ws, wn):
    return pl.pallas_call(
        _tc_mid_body,
        grid=(NBLK,),
        in_specs=[_row_spec(), _row_spec(), _row_spec(), _deg_spec(),
                  _deg_spec(), _full_spec(D, D), _full_spec(D, D)],
        out_specs=_row_spec(),
        out_shape=jax.ShapeDtypeStruct((N, D), jnp.float32),
    )(hs, a0, a1, d0, d1, ws, wn)


def _tc_out(hs, a0, a1, d0, d1, ws, wn, wo, bo):
    return pl.pallas_call(
        _tc_out_body,
        grid=(NBLK,),
        in_specs=[_row_spec(), _row_spec(), _row_spec(), _deg_spec(),
                  _deg_spec(), _full_spec(D, D), _full_spec(D, D),
                  _full_spec(OUT, D), _full_spec(1, OUT)],
        out_specs=pl.BlockSpec((ROWS_BLK, OUT), lambda i: (i, jnp.int32(0))),
        out_shape=jax.ShapeDtypeStruct((N, OUT), jnp.float32),
    )(hs, a0, a1, d0, d1, ws, wn, wo, bo)


def _sc_agg_build(with_deg):
    mesh = plsc.VectorSubcoreMesh(core_axis_name="c", subcore_axis_name="s",
                                  num_cores=NCORES, num_subcores=NSUB)
    out_type = [jax.ShapeDtypeStruct((NCORES * NPAD, D), jnp.float32)]
    scratch = [
        pltpu.VMEM((WIN,), jnp.int32),
        pltpu.VMEM((WIN,), jnp.int32),
        pltpu.VMEM((WIN, D), jnp.float32),
        pltpu.VMEM_SHARED((NPAD, D), jnp.float32),
        pltpu.SemaphoreType.DMA,
    ]
    if with_deg:
        out_type.append(jax.ShapeDtypeStruct((NCORES * NPAD, 16), jnp.float32))
        scratch += [
            pltpu.VMEM((WIN, 16), jnp.float32),
            pltpu.VMEM_SHARED((NPAD, 16), jnp.float32),
        ]

    def body(hs_hbm, dst_hbm, src_hbm, zrow_hbm, zdeg_ones_hbm, agg_out,
             *rest):
        if with_deg:
            deg_out, dst_v, src_v, rows_v, agg_sh, sem, ones_v, deg_sh = rest
        else:
            dst_v, src_v, rows_v, agg_sh, sem = rest
        c = lax.axis_index("c")
        s = lax.axis_index("s")
        wid = s * jnp.int32(NCORES) + c
        rbase = s * jnp.int32(RPT)
        chunks = []
        off = 0
        while off < RPT:
            sz = min(WIN, RPT - off)
            chunks.append((off, sz))
            off += sz
        for off, sz in chunks:
            pltpu.sync_copy(zrow_hbm.at[pl.ds(rbase + jnp.int32(off), sz)],
                            rows_v.at[pl.ds(0, sz)])
            pltpu.sync_copy(rows_v.at[pl.ds(0, sz)],
                            agg_sh.at[pl.ds(rbase + jnp.int32(off), sz)])
            if with_deg:
                pltpu.sync_copy(
                    zdeg_ones_hbm.at[pl.ds(rbase + jnp.int32(off), sz)],
                    ones_v.at[pl.ds(0, sz)])
                pltpu.sync_copy(ones_v.at[pl.ds(0, sz)],
                                deg_sh.at[pl.ds(rbase + jnp.int32(off), sz)])
        if with_deg:
            pltpu.sync_copy(zdeg_ones_hbm.at[pl.ds(NPAD, WIN)], ones_v)
        plsc.subcore_barrier()

        base0 = wid * jnp.int32(EPW)

        @pl.loop(jnp.int32(0), jnp.int32(NWIN))
        def _window(w):
            b = base0 + w * jnp.int32(WIN)
            pltpu.sync_copy(dst_hbm.at[pl.ds(b, WIN)], dst_v)
            pltpu.sync_copy(src_hbm.at[pl.ds(b, WIN)], src_v)
            pltpu.async_copy(hs_hbm.at[dst_v], rows_v, sem).wait()
            pltpu.sync_copy(rows_v, agg_sh.at[src_v], add=True)
            if with_deg:
                pltpu.sync_copy(ones_v, deg_sh.at[src_v], add=True)
        plsc.subcore_barrier()
        obase = c * jnp.int32(NPAD) + rbase
        for off, sz in chunks:
            pltpu.sync_copy(agg_sh.at[pl.ds(rbase + jnp.int32(off), sz)],
                            rows_v.at[pl.ds(0, sz)])
            pltpu.sync_copy(rows_v.at[pl.ds(0, sz)],
                            agg_out.at[pl.ds(obase + jnp.int32(off), sz)])
            if with_deg:
                pltpu.sync_copy(deg_sh.at[pl.ds(rbase + jnp.int32(off), sz)],
                                ones_v.at[pl.ds(0, sz)])
                pltpu.sync_copy(ones_v.at[pl.ds(0, sz)],
                                deg_out.at[pl.ds(obase + jnp.int32(off), sz)])

    return pl.kernel(body, out_type=out_type, mesh=mesh,
                     scratch_types=scratch)


@functools.lru_cache(maxsize=None)
def _sc_agg_cached(with_deg):
    return _sc_agg_build(with_deg)


@jax.jit
def kernel(x, edge_index, W_in, b_in, W_self0, W_neigh0, W_self1, W_neigh1,
           W_out, b_out):
    x = x.astype(jnp.float32)
    W_in = W_in.astype(jnp.float32)
    W_self0 = W_self0.astype(jnp.float32)
    W_neigh0 = W_neigh0.astype(jnp.float32)
    W_self1 = W_self1.astype(jnp.float32)
    W_neigh1 = W_neigh1.astype(jnp.float32)
    W_out = W_out.astype(jnp.float32)
    src = edge_index[0].astype(jnp.int32)
    dst = edge_index[1].astype(jnp.int32)
    pad = EP - E
    pad_i = jnp.arange(pad, dtype=jnp.int32)
    src_p = jnp.concatenate([src, N + (pad_i % (NPAD - N))])
    dst_p = jnp.concatenate([dst, pad_i % 16])

    zrow = jnp.zeros((NPAD, D), jnp.float32)
    zdeg_ones = jnp.concatenate([jnp.zeros((NPAD, 16), jnp.float32),
                                 jnp.ones((WIN, 16), jnp.float32)])

    b_in2 = b_in.astype(jnp.float32).reshape(1, D)
    b_out2 = b_out.astype(jnp.float32).reshape(1, OUT)

    hs0 = _tc_in(x, W_in, b_in2)
    aggf, = _sc_agg_cached(False)(hs0, dst_p, src_p, zrow, zdeg_ones)
    a0, a1 = aggf[:N], aggf[NPAD:NPAD + N]
    ones_mat = jnp.ones((N, D), jnp.float32)
    degf, = _sc_agg_cached(False)(ones_mat, dst_p, src_p, zrow, zdeg_ones)
    d0 = degf[:N, :16]
    d1 = degf[NPAD:NPAD + N, :16]
    hs1 = _tc_mid(hs0, a0, a1, d0, d1, W_self0, W_neigh0)
    aggf1, = _sc_agg_cached(False)(hs1, dst_p, src_p, zrow, zdeg_ones)
    b0, b1 = aggf1[:N], aggf1[NPAD:NPAD + N]
    out = _tc_out(hs1, b0, b1, d0, d1, W_self1, W_neigh1, W_out, b_out2)
    return out.astype(jnp.float64)


# speedup vs baseline: 200.6340x; 1.4941x over previous
import functools

import jax
import jax.numpy as jnp
import numpy as np
from jax import lax
from jax.experimental import pallas as pl
from jax.experimental.pallas import tpu as pltpu
from jax.experimental.pallas import tpu_sc as plsc

N = 10000
E = 320000
D = 128
K = 32
OUT = 128

NCORES = 2
NSUB = 16
NW = NCORES * NSUB
WIN = 128
KBUF = 3
EP = ((E + NW * WIN * KBUF - 1) // (NW * WIN * KBUF)) * (NW * WIN * KBUF)
EPW = EP // NW
NWIN = EPW // WIN
RPT = 632
NPAD = RPT * NSUB

ROWS_BLK = 1000
NBLK = N // ROWS_BLK


def _maxk_block(t):
    b = lax.bitcast_convert_type(t, jnp.int32)
    neg = lax.shift_right_arithmetic(b, jnp.int32(31))
    key = lax.bitwise_xor(b, lax.bitwise_and(neg, jnp.int32(0x7FFFFFFF)))
    sign = jnp.int32(-2147483648)
    t_u = jnp.zeros(t.shape[:1] + (1,), jnp.int32)
    for bit in range(31, -1, -1):
        cand = lax.bitwise_or(t_u, jnp.int32(np.int32(np.uint32(1 << bit))))
        thr = lax.bitwise_xor(cand, sign)
        cnt = jnp.sum((key >= thr).astype(jnp.int32), axis=1, keepdims=True,
                      dtype=jnp.int32)
        t_u = jnp.where(cnt >= K, cand, t_u)
    thresh = lax.bitwise_xor(t_u, sign)
    return t * (key >= thresh).astype(t.dtype)


def _dot_t(a, w):
    return lax.dot_general(a, w, (((1,), (1,)), ((), ())),
                           precision=lax.Precision.HIGHEST,
                           preferred_element_type=jnp.float32)


def _tc_in_body(x_ref, w_ref, b_ref, o_ref):
    o_ref[...] = _maxk_block(_dot_t(x_ref[...], w_ref[...]) + b_ref[...])


def _hagg(a0_ref, a1_ref, d0_ref, d1_ref):
    deg = d0_ref[...][:, 0:1] + d1_ref[...][:, 0:1]
    return (a0_ref[...] + a1_ref[...]) / (deg + jnp.float32(1e-6))


def _tc_mid_body(hs_ref, a0_ref, a1_ref, d0_ref, d1_ref, ws_ref, wn_ref, o_ref):
    t = _dot_t(hs_ref[...], ws_ref[...]) + _dot_t(
        _hagg(a0_ref, a1_ref, d0_ref, d1_ref), wn_ref[...])
    o_ref[...] = _maxk_block(t)


def _tc_out_body(hs_ref, a0_ref, a1_ref, d0_ref, d1_ref, ws_ref, wn_ref,
                 wo_ref, bo_ref, o_ref):
    t = _dot_t(hs_ref[...], ws_ref[...]) + _dot_t(
        _hagg(a0_ref, a1_ref, d0_ref, d1_ref), wn_ref[...])
    o_ref[...] = _dot_t(t, wo_ref[...]) + bo_ref[...]


def _row_spec():
    return pl.BlockSpec((ROWS_BLK, D), lambda i: (i, jnp.int32(0)))


def _deg_spec():
    return pl.BlockSpec((ROWS_BLK, 16), lambda i: (i, jnp.int32(0)))


def _full_spec(r, c):
    return pl.BlockSpec((r, c), lambda i: (jnp.int32(0), jnp.int32(0)))


def _tc_in(x, w, b):
    return pl.pallas_call(
        _tc_in_body,
        grid=(NBLK,),
        in_specs=[_row_spec(), _full_spec(D, D), _full_spec(1, D)],
        out_specs=_row_spec(),
        out_shape=jax.ShapeDtypeStruct((N, D), jnp.float32),
    )(x, w, b)


def _tc_mid(hs, a0, a1, d0, d1, ws, wn):
    return pl.pallas_call(
        _tc_mid_body,
        grid=(NBLK,),
        in_specs=[_row_spec(), _row_spec(), _row_spec(), _deg_spec(),
                  _deg_spec(), _full_spec(D, D), _full_spec(D, D)],
        out_specs=_row_spec(),
        out_shape=jax.ShapeDtypeStruct((N, D), jnp.float32),
    )(hs, a0, a1, d0, d1, ws, wn)


def _tc_out(hs, a0, a1, d0, d1, ws, wn, wo, bo):
    return pl.pallas_call(
        _tc_out_body,
        grid=(NBLK,),
        in_specs=[_row_spec(), _row_spec(), _row_spec(), _deg_spec(),
                  _deg_spec(), _full_spec(D, D), _full_spec(D, D),
                  _full_spec(OUT, D), _full_spec(1, OUT)],
        out_specs=pl.BlockSpec((ROWS_BLK, OUT), lambda i: (i, jnp.int32(0))),
        out_shape=jax.ShapeDtypeStruct((N, OUT), jnp.float32),
    )(hs, a0, a1, d0, d1, ws, wn, wo, bo)


def _sc_agg_build(gather):
    mesh = plsc.VectorSubcoreMesh(core_axis_name="c", subcore_axis_name="s",
                                  num_cores=NCORES, num_subcores=NSUB)
    out_type = [jax.ShapeDtypeStruct((NCORES * NPAD, D), jnp.float32)]
    nrows = KBUF if gather else 1
    scratch = (
        [pltpu.VMEM((WIN,), jnp.int32) for _ in range(KBUF)]
        + [pltpu.VMEM((WIN,), jnp.int32) for _ in range(KBUF)]
        + [pltpu.VMEM((WIN, D), jnp.float32) for _ in range(nrows)]
        + [pltpu.VMEM_SHARED((NPAD, D), jnp.float32)]
        + [pltpu.SemaphoreType.DMA]
        + [pltpu.SemaphoreType.DMA for _ in range(KBUF)]
        + [pltpu.SemaphoreType.DMA for _ in range(KBUF)]
    )

    def body(hs_hbm, dst_hbm, src_hbm, zrow_hbm, agg_out, *rest):
        rest = list(rest)
        dst_v = [rest.pop(0) for _ in range(KBUF)]
        src_v = [rest.pop(0) for _ in range(KBUF)]
        rows_v = [rest.pop(0) for _ in range(nrows)]
        agg_sh = rest.pop(0)
        isem = rest.pop(0)
        gsem = [rest.pop(0) for _ in range(KBUF)]
        ssem = [rest.pop(0) for _ in range(KBUF)]
        c = lax.axis_index("c")
        s = lax.axis_index("s")
        wid = s * jnp.int32(NCORES) + c
        rbase = s * jnp.int32(RPT)
        chunks = []
        off = 0
        while off < RPT:
            sz = min(WIN, RPT - off)
            chunks.append((off, sz))
            off += sz
        for off, sz in chunks:
            pltpu.sync_copy(zrow_hbm.at[pl.ds(rbase + jnp.int32(off), sz)],
                            rows_v[0].at[pl.ds(0, sz)])
            pltpu.sync_copy(rows_v[0].at[pl.ds(0, sz)],
                            agg_sh.at[pl.ds(rbase + jnp.int32(off), sz)])
        if not gather:
            pltpu.sync_copy(hs_hbm.at[pl.ds(0, WIN)], rows_v[0])
        plsc.subcore_barrier()

        base0 = wid * jnp.int32(EPW)

        @pl.loop(jnp.int32(0), jnp.int32(NWIN // KBUF))
        def _batch(i):
            b0 = base0 + i * jnp.int32(KBUF * WIN)
            idescs = []
            for k in range(KBUF):
                bk = b0 + jnp.int32(k * WIN)
                if gather:
                    idescs.append(pltpu.async_copy(
                        dst_hbm.at[pl.ds(bk, WIN)], dst_v[k], isem))
                idescs.append(pltpu.async_copy(
                    src_hbm.at[pl.ds(bk, WIN)], src_v[k], isem))
            for d in idescs:
                d.wait()
            sdescs = []
            if gather:
                gdescs = [pltpu.async_copy(hs_hbm.at[dst_v[k]], rows_v[k],
                                           gsem[k]) for k in range(KBUF)]
                for k in range(KBUF):
                    gdescs[k].wait()
                    sdescs.append(pltpu.async_copy(
                        rows_v[k], agg_sh.at[src_v[k]], ssem[k], add=True))
            else:
                for k in range(KBUF):
                    sdescs.append(pltpu.async_copy(
                        rows_v[0], agg_sh.at[src_v[k]], ssem[k], add=True))
            for d in sdescs:
                d.wait()

        plsc.subcore_barrier()
        obase = c * jnp.int32(NPAD) + rbase
        for off, sz in chunks:
            pltpu.sync_copy(agg_sh.at[pl.ds(rbase + jnp.int32(off), sz)],
                            rows_v[0].at[pl.ds(0, sz)])
            pltpu.sync_copy(rows_v[0].at[pl.ds(0, sz)],
                            agg_out.at[pl.ds(obase + jnp.int32(off), sz)])

    return pl.kernel(body, out_type=out_type, mesh=mesh,
                     scratch_types=scratch)


@functools.lru_cache(maxsize=None)
def _sc_agg_cached(gather):
    return _sc_agg_build(gather)


@jax.jit
def kernel(x, edge_index, W_in, b_in, W_self0, W_neigh0, W_self1, W_neigh1,
           W_out, b_out):
    x = x.astype(jnp.float32)
    W_in = W_in.astype(jnp.float32)
    W_self0 = W_self0.astype(jnp.float32)
    W_neigh0 = W_neigh0.astype(jnp.float32)
    W_self1 = W_self1.astype(jnp.float32)
    W_neigh1 = W_neigh1.astype(jnp.float32)
    W_out = W_out.astype(jnp.float32)
    src = edge_index[0].astype(jnp.int32)
    dst = edge_index[1].astype(jnp.int32)
    pad = EP - E
    pad_i = jnp.arange(pad, dtype=jnp.int32)
    src_p = jnp.concatenate([src, N + (pad_i % (NPAD - N))])
    dst_p = jnp.concatenate([dst, pad_i % 16])

    zrow = jnp.zeros((NPAD, D), jnp.float32)

    b_in2 = b_in.astype(jnp.float32).reshape(1, D)
    b_out2 = b_out.astype(jnp.float32).reshape(1, OUT)

    hs0 = _tc_in(x, W_in, b_in2)
    aggf, = _sc_agg_cached(True)(hs0, dst_p, src_p, zrow)
    a0, a1 = aggf[:N], aggf[NPAD:NPAD + N]
    ones_blk = jnp.ones((WIN, D), jnp.float32)
    degf, = _sc_agg_cached(False)(ones_blk, dst_p, src_p, zrow)
    d0 = degf[:N, :16]
    d1 = degf[NPAD:NPAD + N, :16]
    hs1 = _tc_mid(hs0, a0, a1, d0, d1, W_self0, W_neigh0)
    aggf1, = _sc_agg_cached(True)(hs1, dst_p, src_p, zrow)
    b0, b1 = aggf1[:N], aggf1[NPAD:NPAD + N]
    out = _tc_out(hs1, b0, b1, d0, d1, W_self1, W_neigh1, W_out, b_out2)
    return out.astype(jnp.float64)


# speedup vs baseline: 200.8654x; 1.0012x over previous
import functools

import jax
import jax.numpy as jnp
import numpy as np
from jax import lax
from jax.experimental import pallas as pl
from jax.experimental.pallas import tpu as pltpu
from jax.experimental.pallas import tpu_sc as plsc

N = 10000
E = 320000
D = 128
K = 32
OUT = 128

NCORES = 2
NSUB = 16
NW = NCORES * NSUB
WIN = 128
KBUF = 3
EP = ((E + NW * WIN * KBUF - 1) // (NW * WIN * KBUF)) * (NW * WIN * KBUF)
EPW = EP // NW
NWIN = EPW // WIN
RPT = 632
NPAD = RPT * NSUB

ROWS_BLK = 1000
NBLK = N // ROWS_BLK


def _maxk_block(t):
    b = lax.bitcast_convert_type(t, jnp.int32)
    neg = lax.shift_right_arithmetic(b, jnp.int32(31))
    key = lax.bitwise_xor(b, lax.bitwise_and(neg, jnp.int32(0x7FFFFFFF)))
    sign = jnp.int32(-2147483648)
    t_u = jnp.zeros(t.shape[:1] + (1,), jnp.int32)
    for bit in range(31, -1, -1):
        cand = lax.bitwise_or(t_u, jnp.int32(np.int32(np.uint32(1 << bit))))
        thr = lax.bitwise_xor(cand, sign)
        cnt = jnp.sum((key >= thr).astype(jnp.int32), axis=1, keepdims=True,
                      dtype=jnp.int32)
        t_u = jnp.where(cnt >= K, cand, t_u)
    thresh = lax.bitwise_xor(t_u, sign)
    return t * (key >= thresh).astype(t.dtype)


def _dot_t(a, w):
    return lax.dot_general(a, w, (((1,), (1,)), ((), ())),
                           precision=lax.Precision.HIGHEST,
                           preferred_element_type=jnp.float32)


def _tc_in_body(x_ref, w_ref, b_ref, o_ref):
    o_ref[...] = _maxk_block(_dot_t(x_ref[...], w_ref[...]) + b_ref[...])


def _hagg(a0_ref, a1_ref, d0_ref, d1_ref):
    deg = d0_ref[...][:, 0:1] + d1_ref[...][:, 0:1]
    return (a0_ref[...] + a1_ref[...]) / (deg + jnp.float32(1e-6))


def _tc_mid_body(hs_ref, a0_ref, a1_ref, d0_ref, d1_ref, ws_ref, wn_ref, o_ref):
    t = _dot_t(hs_ref[...], ws_ref[...]) + _dot_t(
        _hagg(a0_ref, a1_ref, d0_ref, d1_ref), wn_ref[...])
    o_ref[...] = _maxk_block(t)


def _tc_out_body(hs_ref, a0_ref, a1_ref, d0_ref, d1_ref, ws_ref, wn_ref,
                 wo_ref, bo_ref, o_ref):
    t = _dot_t(hs_ref[...], ws_ref[...]) + _dot_t(
        _hagg(a0_ref, a1_ref, d0_ref, d1_ref), wn_ref[...])
    o_ref[...] = _dot_t(t, wo_ref[...]) + bo_ref[...]


def _row_spec():
    return pl.BlockSpec((ROWS_BLK, D), lambda i: (i, jnp.int32(0)))


def _deg_spec():
    return pl.BlockSpec((ROWS_BLK, 16), lambda i: (i, jnp.int32(0)))


def _full_spec(r, c):
    return pl.BlockSpec((r, c), lambda i: (jnp.int32(0), jnp.int32(0)))


def _tc_in(x, w, b):
    return pl.pallas_call(
        _tc_in_body,
        grid=(NBLK,),
        in_specs=[_row_spec(), _full_spec(D, D), _full_spec(1, D)],
        out_specs=_row_spec(),
        out_shape=jax.ShapeDtypeStruct((N, D), jnp.float32),
    )(x, w, b)


def _tc_mid(hs, a0, a1, d0, d1, ws, wn):
    return pl.pallas_call(
        _tc_mid_body,
        grid=(NBLK,),
        in_specs=[_row_spec(), _row_spec(), _row_spec(), _deg_spec(),
                  _deg_spec(), _full_spec(D, D), _full_spec(D, D)],
        out_specs=_row_spec(),
        out_shape=jax.ShapeDtypeStruct((N, D), jnp.float32),
    )(hs, a0, a1, d0, d1, ws, wn)


def _tc_out(hs, a0, a1, d0, d1, ws, wn, wo, bo):
    return pl.pallas_call(
        _tc_out_body,
        grid=(NBLK,),
        in_specs=[_row_spec(), _row_spec(), _row_spec(), _deg_spec(),
                  _deg_spec(), _full_spec(D, D), _full_spec(D, D),
                  _full_spec(OUT, D), _full_spec(1, OUT)],
        out_specs=pl.BlockSpec((ROWS_BLK, OUT), lambda i: (i, jnp.int32(0))),
        out_shape=jax.ShapeDtypeStruct((N, OUT), jnp.float32),
    )(hs, a0, a1, d0, d1, ws, wn, wo, bo)


def _sc_agg_build(gather):
    mesh = plsc.VectorSubcoreMesh(core_axis_name="c", subcore_axis_name="s",
                                  num_cores=NCORES, num_subcores=NSUB)
    out_type = [jax.ShapeDtypeStruct((NCORES * NPAD, D), jnp.float32)]
    nrows = KBUF if gather else 1
    scratch = (
        [pltpu.VMEM((WIN,), jnp.int32) for _ in range(KBUF)]
        + [pltpu.VMEM((WIN,), jnp.int32) for _ in range(KBUF)]
        + [pltpu.VMEM((WIN, D), jnp.float32) for _ in range(nrows)]
        + [pltpu.VMEM_SHARED((NPAD, D), jnp.float32)]
        + [pltpu.SemaphoreType.DMA]
        + [pltpu.SemaphoreType.DMA for _ in range(KBUF)]
        + [pltpu.SemaphoreType.DMA for _ in range(KBUF)]
    )

    def body(hs_hbm, dst_hbm, src_hbm, zrow_hbm, agg_out, *rest):
        rest = list(rest)
        dst_v = [rest.pop(0) for _ in range(KBUF)]
        src_v = [rest.pop(0) for _ in range(KBUF)]
        rows_v = [rest.pop(0) for _ in range(nrows)]
        agg_sh = rest.pop(0)
        isem = rest.pop(0)
        gsem = [rest.pop(0) for _ in range(KBUF)]
        ssem = [rest.pop(0) for _ in range(KBUF)]
        c = lax.axis_index("c")
        s = lax.axis_index("s")
        wid = s * jnp.int32(NCORES) + c
        rbase = s * jnp.int32(RPT)
        chunks = []
        off = 0
        while off < RPT:
            sz = min(WIN, RPT - off)
            chunks.append((off, sz))
            off += sz
        for off, sz in chunks:
            pltpu.sync_copy(zrow_hbm.at[pl.ds(rbase + jnp.int32(off), sz)],
                            rows_v[0].at[pl.ds(0, sz)])
            pltpu.sync_copy(rows_v[0].at[pl.ds(0, sz)],
                            agg_sh.at[pl.ds(rbase + jnp.int32(off), sz)])
        if not gather:
            pltpu.sync_copy(hs_hbm.at[pl.ds(0, WIN)], rows_v[0])
        plsc.subcore_barrier()

        base0 = wid * jnp.int32(EPW)

        def _drain_scatters():
            for k in range(KBUF):
                pltpu.make_async_copy(rows_v[k if gather else 0],
                                      agg_sh.at[src_v[k]], ssem[k]).wait()

        @pl.loop(jnp.int32(0), jnp.int32(NWIN // KBUF))
        def _batch(i):
            @pl.when(i > jnp.int32(0))
            def _():
                _drain_scatters()
            b0 = base0 + i * jnp.int32(KBUF * WIN)
            idescs = []
            for k in range(KBUF):
                bk = b0 + jnp.int32(k * WIN)
                if gather:
                    idescs.append(pltpu.async_copy(
                        dst_hbm.at[pl.ds(bk, WIN)], dst_v[k], isem))
                idescs.append(pltpu.async_copy(
                    src_hbm.at[pl.ds(bk, WIN)], src_v[k], isem))
            for d in idescs:
                d.wait()
            if gather:
                gdescs = [pltpu.async_copy(hs_hbm.at[dst_v[k]], rows_v[k],
                                           gsem[k]) for k in range(KBUF)]
                for k in range(KBUF):
                    gdescs[k].wait()
                    pltpu.async_copy(
                        rows_v[k], agg_sh.at[src_v[k]], ssem[k], add=True)
            else:
                for k in range(KBUF):
                    pltpu.async_copy(
                        rows_v[0], agg_sh.at[src_v[k]], ssem[k], add=True)

        _drain_scatters()
        plsc.subcore_barrier()
        obase = c * jnp.int32(NPAD) + rbase
        for off, sz in chunks:
            pltpu.sync_copy(agg_sh.at[pl.ds(rbase + jnp.int32(off), sz)],
                            rows_v[0].at[pl.ds(0, sz)])
            pltpu.sync_copy(rows_v[0].at[pl.ds(0, sz)],
                            agg_out.at[pl.ds(obase + jnp.int32(off), sz)])

    return pl.kernel(body, out_type=out_type, mesh=mesh,
                     scratch_types=scratch)


@functools.lru_cache(maxsize=None)
def _sc_agg_cached(gather):
    return _sc_agg_build(gather)


@jax.jit
def kernel(x, edge_index, W_in, b_in, W_self0, W_neigh0, W_self1, W_neigh1,
           W_out, b_out):
    x = x.astype(jnp.float32)
    W_in = W_in.astype(jnp.float32)
    W_self0 = W_self0.astype(jnp.float32)
    W_neigh0 = W_neigh0.astype(jnp.float32)
    W_self1 = W_self1.astype(jnp.float32)
    W_neigh1 = W_neigh1.astype(jnp.float32)
    W_out = W_out.astype(jnp.float32)
    src = edge_index[0].astype(jnp.int32)
    dst = edge_index[1].astype(jnp.int32)
    pad = EP - E
    pad_i = jnp.arange(pad, dtype=jnp.int32)
    src_p = jnp.concatenate([src, N + (pad_i % (NPAD - N))])
    dst_p = jnp.concatenate([dst, pad_i % 16])

    zrow = jnp.zeros((NPAD, D), jnp.float32)

    b_in2 = b_in.astype(jnp.float32).reshape(1, D)
    b_out2 = b_out.astype(jnp.float32).reshape(1, OUT)

    hs0 = _tc_in(x, W_in, b_in2)
    aggf, = _sc_agg_cached(True)(hs0, dst_p, src_p, zrow)
    a0, a1 = aggf[:N], aggf[NPAD:NPAD + N]
    ones_blk = jnp.ones((WIN, D), jnp.float32)
    degf, = _sc_agg_cached(False)(ones_blk, dst_p, src_p, zrow)
    d0 = degf[:N, :16]
    d1 = degf[NPAD:NPAD + N, :16]
    hs1 = _tc_mid(hs0, a0, a1, d0, d1, W_self0, W_neigh0)
    aggf1, = _sc_agg_cached(True)(hs1, dst_p, src_p, zrow)
    b0, b1 = aggf1[:N], aggf1[NPAD:NPAD + N]
    out = _tc_out(hs1, b0, b1, d0, d1, W_self1, W_neigh1, W_out, b_out2)
    return out.astype(jnp.float64)


# speedup vs baseline: 202.8316x; 1.0098x over previous
import functools

import jax
import jax.numpy as jnp
import numpy as np
from jax import lax
from jax.experimental import pallas as pl
from jax.experimental.pallas import tpu as pltpu
from jax.experimental.pallas import tpu_sc as plsc

N = 10000
E = 320000
D = 128
K = 32
OUT = 128

NCORES = 2
NSUB = 16
NW = NCORES * NSUB
WIN = 128
KBUF = 3
EP = ((E + NW * WIN * KBUF - 1) // (NW * WIN * KBUF)) * (NW * WIN * KBUF)
EPW = EP // NW
NWIN = EPW // WIN
RPT = 632
NPAD = RPT * NSUB

ROWS_BLK = 1000
NBLK = N // ROWS_BLK


def _maxk_block(t):
    b = lax.bitcast_convert_type(t, jnp.int32)
    neg = lax.shift_right_arithmetic(b, jnp.int32(31))
    key = lax.bitwise_xor(b, lax.bitwise_and(neg, jnp.int32(0x7FFFFFFF)))
    sign = jnp.int32(-2147483648)
    ones_cnt = jnp.ones((D, 8), jnp.float32)
    t_u = jnp.zeros(t.shape[:1] + (1,), jnp.int32)
    for bit in range(31, -1, -1):
        cand = lax.bitwise_or(t_u, jnp.int32(np.int32(np.uint32(1 << bit))))
        thr = lax.bitwise_xor(cand, sign)
        m = (key >= thr).astype(jnp.float32)
        cnt = lax.dot_general(m, ones_cnt, (((1,), (0,)), ((), ())),
                              preferred_element_type=jnp.float32)[:, :1]
        t_u = jnp.where(cnt >= jnp.float32(K), cand, t_u)
    thresh = lax.bitwise_xor(t_u, sign)
    return t * (key >= thresh).astype(t.dtype)


def _dot_t(a, w):
    return lax.dot_general(a, w, (((1,), (1,)), ((), ())),
                           precision=lax.Precision.HIGHEST,
                           preferred_element_type=jnp.float32)


def _tc_in_body(x_ref, w_ref, b_ref, o_ref):
    o_ref[...] = _maxk_block(_dot_t(x_ref[...], w_ref[...]) + b_ref[...])


def _hagg(a0_ref, a1_ref, d0_ref, d1_ref):
    deg = d0_ref[...][:, 0:1] + d1_ref[...][:, 0:1]
    return (a0_ref[...] + a1_ref[...]) / (deg + jnp.float32(1e-6))


def _tc_mid_body(hs_ref, a0_ref, a1_ref, d0_ref, d1_ref, ws_ref, wn_ref, o_ref):
    t = _dot_t(hs_ref[...], ws_ref[...]) + _dot_t(
        _hagg(a0_ref, a1_ref, d0_ref, d1_ref), wn_ref[...])
    o_ref[...] = _maxk_block(t)


def _tc_out_body(hs_ref, a0_ref, a1_ref, d0_ref, d1_ref, ws_ref, wn_ref,
                 wo_ref, bo_ref, o_ref):
    t = _dot_t(hs_ref[...], ws_ref[...]) + _dot_t(
        _hagg(a0_ref, a1_ref, d0_ref, d1_ref), wn_ref[...])
    o_ref[...] = _dot_t(t, wo_ref[...]) + bo_ref[...]


def _row_spec():
    return pl.BlockSpec((ROWS_BLK, D), lambda i: (i, jnp.int32(0)))


def _deg_spec():
    return pl.BlockSpec((ROWS_BLK, 16), lambda i: (i, jnp.int32(0)))


def _full_spec(r, c):
    return pl.BlockSpec((r, c), lambda i: (jnp.int32(0), jnp.int32(0)))


def _tc_in(x, w, b):
    return pl.pallas_call(
        _tc_in_body,
        grid=(NBLK,),
        in_specs=[_row_spec(), _full_spec(D, D), _full_spec(1, D)],
        out_specs=_row_spec(),
        out_shape=jax.ShapeDtypeStruct((N, D), jnp.float32),
    )(x, w, b)


def _tc_mid(hs, a0, a1, d0, d1, ws, wn):
    return pl.pallas_call(
        _tc_mid_body,
        grid=(NBLK,),
        in_specs=[_row_spec(), _row_spec(), _row_spec(), _deg_spec(),
                  _deg_spec(), _full_spec(D, D), _full_spec(D, D)],
        out_specs=_row_spec(),
        out_shape=jax.ShapeDtypeStruct((N, D), jnp.float32),
    )(hs, a0, a1, d0, d1, ws, wn)


def _tc_out(hs, a0, a1, d0, d1, ws, wn, wo, bo):
    return pl.pallas_call(
        _tc_out_body,
        grid=(NBLK,),
        in_specs=[_row_spec(), _row_spec(), _row_spec(), _deg_spec(),
                  _deg_spec(), _full_spec(D, D), _full_spec(D, D),
                  _full_spec(OUT, D), _full_spec(1, OUT)],
        out_specs=pl.BlockSpec((ROWS_BLK, OUT), lambda i: (i, jnp.int32(0))),
        out_shape=jax.ShapeDtypeStruct((N, OUT), jnp.float32),
    )(hs, a0, a1, d0, d1, ws, wn, wo, bo)


def _sc_agg_build(gather):
    mesh = plsc.VectorSubcoreMesh(core_axis_name="c", subcore_axis_name="s",
                                  num_cores=NCORES, num_subcores=NSUB)
    out_type = [jax.ShapeDtypeStruct((NCORES * NPAD, D), jnp.float32)]
    nrows = KBUF if gather else 1
    scratch = (
        [pltpu.VMEM((WIN,), jnp.int32) for _ in range(KBUF)]
        + [pltpu.VMEM((WIN,), jnp.int32) for _ in range(KBUF)]
        + [pltpu.VMEM((WIN, D), jnp.float32) for _ in range(nrows)]
        + [pltpu.VMEM_SHARED((NPAD, D), jnp.float32)]
        + [pltpu.SemaphoreType.DMA]
        + [pltpu.SemaphoreType.DMA for _ in range(KBUF)]
        + [pltpu.SemaphoreType.DMA for _ in range(KBUF)]
    )

    def body(hs_hbm, dst_hbm, src_hbm, zrow_hbm, agg_out, *rest):
        rest = list(rest)
        dst_v = [rest.pop(0) for _ in range(KBUF)]
        src_v = [rest.pop(0) for _ in range(KBUF)]
        rows_v = [rest.pop(0) for _ in range(nrows)]
        agg_sh = rest.pop(0)
        isem = rest.pop(0)
        gsem = [rest.pop(0) for _ in range(KBUF)]
        ssem = [rest.pop(0) for _ in range(KBUF)]
        c = lax.axis_index("c")
        s = lax.axis_index("s")
        wid = s * jnp.int32(NCORES) + c
        rbase = s * jnp.int32(RPT)
        chunks = []
        off = 0
        while off < RPT:
            sz = min(WIN, RPT - off)
            chunks.append((off, sz))
            off += sz
        for off, sz in chunks:
            pltpu.sync_copy(zrow_hbm.at[pl.ds(rbase + jnp.int32(off), sz)],
                            rows_v[0].at[pl.ds(0, sz)])
            pltpu.sync_copy(rows_v[0].at[pl.ds(0, sz)],
                            agg_sh.at[pl.ds(rbase + jnp.int32(off), sz)])
        if not gather:
            pltpu.sync_copy(hs_hbm.at[pl.ds(0, WIN)], rows_v[0])
        plsc.subcore_barrier()

        base0 = wid * jnp.int32(EPW)

        def _drain_scatters():
            for k in range(KBUF):
                pltpu.make_async_copy(rows_v[k if gather else 0],
                                      agg_sh.at[src_v[k]], ssem[k]).wait()

        @pl.loop(jnp.int32(0), jnp.int32(NWIN // KBUF))
        def _batch(i):
            @pl.when(i > jnp.int32(0))
            def _():
                _drain_scatters()
            b0 = base0 + i * jnp.int32(KBUF * WIN)
            idescs = []
            for k in range(KBUF):
                bk = b0 + jnp.int32(k * WIN)
                if gather:
                    idescs.append(pltpu.async_copy(
                        dst_hbm.at[pl.ds(bk, WIN)], dst_v[k], isem))
                idescs.append(pltpu.async_copy(
                    src_hbm.at[pl.ds(bk, WIN)], src_v[k], isem))
            for d in idescs:
                d.wait()
            if gather:
                gdescs = [pltpu.async_copy(hs_hbm.at[dst_v[k]], rows_v[k],
                                           gsem[k]) for k in range(KBUF)]
                for k in range(KBUF):
                    gdescs[k].wait()
                    pltpu.async_copy(
                        rows_v[k], agg_sh.at[src_v[k]], ssem[k], add=True)
            else:
                for k in range(KBUF):
                    pltpu.async_copy(
                        rows_v[0], agg_sh.at[src_v[k]], ssem[k], add=True)

        _drain_scatters()
        plsc.subcore_barrier()
        obase = c * jnp.int32(NPAD) + rbase
        for off, sz in chunks:
            pltpu.sync_copy(agg_sh.at[pl.ds(rbase + jnp.int32(off), sz)],
                            rows_v[0].at[pl.ds(0, sz)])
            pltpu.sync_copy(rows_v[0].at[pl.ds(0, sz)],
                            agg_out.at[pl.ds(obase + jnp.int32(off), sz)])

    return pl.kernel(body, out_type=out_type, mesh=mesh,
                     scratch_types=scratch)


@functools.lru_cache(maxsize=None)
def _sc_agg_cached(gather):
    return _sc_agg_build(gather)


@jax.jit
def kernel(x, edge_index, W_in, b_in, W_self0, W_neigh0, W_self1, W_neigh1,
           W_out, b_out):
    x = x.astype(jnp.float32)
    W_in = W_in.astype(jnp.float32)
    W_self0 = W_self0.astype(jnp.float32)
    W_neigh0 = W_neigh0.astype(jnp.float32)
    W_self1 = W_self1.astype(jnp.float32)
    W_neigh1 = W_neigh1.astype(jnp.float32)
    W_out = W_out.astype(jnp.float32)
    src = edge_index[0].astype(jnp.int32)
    dst = edge_index[1].astype(jnp.int32)
    pad = EP - E
    pad_i = jnp.arange(pad, dtype=jnp.int32)
    src_p = jnp.concatenate([src, N + (pad_i % (NPAD - N))])
    dst_p = jnp.concatenate([dst, pad_i % 16])

    zrow = jnp.zeros((NPAD, D), jnp.float32)

    b_in2 = b_in.astype(jnp.float32).reshape(1, D)
    b_out2 = b_out.astype(jnp.float32).reshape(1, OUT)

    hs0 = _tc_in(x, W_in, b_in2)
    aggf, = _sc_agg_cached(True)(hs0, dst_p, src_p, zrow)
    a0, a1 = aggf[:N], aggf[NPAD:NPAD + N]
    ones_blk = jnp.ones((WIN, D), jnp.float32)
    degf, = _sc_agg_cached(False)(ones_blk, dst_p, src_p, zrow)
    d0 = degf[:N, :16]
    d1 = degf[NPAD:NPAD + N, :16]
    hs1 = _tc_mid(hs0, a0, a1, d0, d1, W_self0, W_neigh0)
    aggf1, = _sc_agg_cached(True)(hs1, dst_p, src_p, zrow)
    b0, b1 = aggf1[:N], aggf1[NPAD:NPAD + N]
    out = _tc_out(hs1, b0, b1, d0, d1, W_self1, W_neigh1, W_out, b_out2)
    return out.astype(jnp.float64)
